# baseline (speedup 1.0000x reference)
import functools

import jax
import jax.numpy as jnp
from jax import lax
from jax.experimental import pallas as pl
from jax.experimental.pallas import tpu as pltpu
from jax.experimental.pallas import tpu_sc as plsc

NUM_ROWS = 16384
VOCAB_DIM = 100000
EMBED_DIM = 64
NNZ_TOTAL = 1048576

NCORES = 2
NSUB = 16
HALF_ROWS = NUM_ROWS // NCORES
TRASH_ROW = HALF_ROWS
CHUNK = 256
NCHUNKS = NNZ_TOTAL // CHUNK
ISEG = 128
NSEG = CHUNK // ISEG
ROWS_PER_TILE = HALF_ROWS // NSUB


def _tc_split_count(row2):
    def body(r_ref, s_ref):
        cnt = jnp.sum((r_ref[...] < HALF_ROWS).astype(jnp.int32))
        s_ref[...] = jnp.full((8, 128), cnt, jnp.int32)

    return pl.pallas_call(
        body,
        out_shape=jax.ShapeDtypeStruct((8, 128), jnp.int32),
    )(row2)


def _sc_aggregate(table, values, row2, col2, split):
    mesh = plsc.VectorSubcoreMesh(core_axis_name="c", subcore_axis_name="s")

    @functools.partial(
        pl.kernel,
        out_type=jax.ShapeDtypeStruct((NUM_ROWS, EMBED_DIM), jnp.float32),
        mesh=mesh,
        compiler_params=pltpu.CompilerParams(use_tc_tiling_on_sc=False),
        scratch_types=[
            pltpu.VMEM_SHARED((HALF_ROWS + 8, EMBED_DIM), jnp.float32),
            pltpu.VMEM((8, 128), jnp.int32),
            [pltpu.VMEM((NSEG, ISEG), jnp.int32)] * 4,
            [pltpu.VMEM((NSEG, ISEG), jnp.int32)] * 4,
            [pltpu.VMEM((NSEG, ISEG), jnp.int32)] * 4,
            [pltpu.VMEM((CHUNK,), jnp.float32)] * 4,
            [pltpu.VMEM((CHUNK, EMBED_DIM), jnp.float32)] * 4,
            [pltpu.SemaphoreType.DMA] * 4,
            [pltpu.SemaphoreType.DMA] * 4,
            [pltpu.SemaphoreType.DMA] * 4,
        ],
    )
    def body(table_hbm, vals_hbm, row_hbm, col_hbm, split_hbm, out_hbm,
             acc, split_v, col_v, row_v, sidx_v, vals_v, rows_v,
             semA, semG, semS):
        cid = lax.axis_index("c")
        sid = lax.axis_index("s")

        zeros = jnp.zeros((16,), jnp.float32)

        @plsc.parallel_loop(0, CHUNK, 1)
        def zero_body(t):
            for k in range(EMBED_DIM // 16):
                rows_v[0][t, pl.ds(k * 16, 16)] = zeros

        for h in range(ROWS_PER_TILE // CHUNK):
            pltpu.sync_copy(
                rows_v[0],
                acc.at[pl.ds(sid * ROWS_PER_TILE + h * CHUNK, CHUNK)])

        @pl.when(sid == 0)
        def _():
            pltpu.sync_copy(rows_v[0].at[pl.ds(0, 8)],
                            acc.at[pl.ds(HALF_ROWS, 8)])

        plsc.subcore_barrier()

        pltpu.sync_copy(split_hbm, split_v)
        split = split_v[0, pl.ds(0, 16)][0]
        cb = split // CHUNK
        lo = cb * cid
        hi = jnp.where(cid == 0, jnp.minimum(cb + 1, NCHUNKS), NCHUNKS)
        n = hi - lo
        niter = jnp.maximum(0, (n - sid + NSUB - 1) // NSUB)
        row_base = cid * HALF_ROWS
        lane_splats = [jnp.full((16,), l, jnp.int32) for l in range(16)]

        def chunk_of(i):
            return lo + sid + i * NSUB

        def issue_aux(i, b):
            ch = chunk_of(i)
            base = pl.multiple_of(ch * CHUNK, CHUNK)
            seg_base = pl.multiple_of(ch * NSEG, NSEG)
            pltpu.async_copy(vals_hbm.at[pl.ds(base, CHUNK)], vals_v[b],
                             semA[b])
            pltpu.async_copy(col_hbm.at[pl.ds(seg_base, NSEG)], col_v[b],
                             semA[b])
            pltpu.async_copy(row_hbm.at[pl.ds(seg_base, NSEG)], row_v[b],
                             semA[b])

        def wait_aux(b):
            pltpu.make_async_copy(vals_hbm.at[pl.ds(0, CHUNK)], vals_v[b],
                                  semA[b]).wait()
            pltpu.make_async_copy(col_hbm.at[pl.ds(0, NSEG)], col_v[b],
                                  semA[b]).wait()
            pltpu.make_async_copy(row_hbm.at[pl.ds(0, NSEG)], row_v[b],
                                  semA[b]).wait()

        def issue_gathers(b):
            for j in range(NSEG):
                pltpu.async_copy(table_hbm.at[col_v[b].at[j]],
                                 rows_v[b].at[pl.ds(j * ISEG, ISEG)],
                                 semG[b])

        def wait_gathers(b):
            for j in range(NSEG):
                pltpu.make_async_copy(table_hbm.at[col_v[b].at[j]],
                                      rows_v[b].at[pl.ds(j * ISEG, ISEG)],
                                      semG[b]).wait()

        def issue_scatters(b):
            for j in range(NSEG):
                pltpu.async_copy(rows_v[b].at[pl.ds(j * ISEG, ISEG)],
                                 acc.at[sidx_v[b].at[j]],
                                 semS[b], add=True)

        def wait_scatters(b):
            for j in range(NSEG):
                pltpu.make_async_copy(rows_v[b].at[pl.ds(j * ISEG, ISEG)],
                                      acc.at[sidx_v[b].at[j]],
                                      semS[b]).wait()

        def compute(b):
            for j in range(NSEG):
                for g in range(ISEG // 16):
                    r = row_v[b][j, pl.ds(g * 16, 16)]
                    loc = r - row_base
                    ok = (loc >= 0) & (loc < HALF_ROWS)
                    sidx_v[b][j, pl.ds(g * 16, 16)] = jnp.where(
                        ok, loc, TRASH_ROW)

            @plsc.parallel_loop(0, CHUNK, 16, unroll=2)
            def mul_body(t0):
                vals16 = vals_v[b][pl.ds(pl.multiple_of(t0, 16), 16)]
                for l in range(16):
                    v = vals16.at[lane_splats[l]].get(
                        mode="promise_in_bounds")
                    for k in range(EMBED_DIM // 16):
                        sl = rows_v[b][t0 + l, pl.ds(k * 16, 16)]
                        rows_v[b][t0 + l, pl.ds(k * 16, 16)] = sl * v

        for c in range(4):
            @pl.when(niter > c)
            def _(c=c):
                issue_aux(c, c)

        for c in range(3):
            @pl.when(niter > c)
            def _(c=c):
                wait_aux(c)
                issue_gathers(c)

        def quad_body(t, _):
            i0 = t * 4

            def phase(i, s0, s3):
                @pl.when(i + 3 < niter)
                def _():
                    @pl.when(i >= 1)
                    def _():
                        wait_scatters(s3)

                    wait_aux(s3)
                    issue_gathers(s3)

                @pl.when(i < niter)
                def _():
                    wait_gathers(s0)
                    compute(s0)
                    issue_scatters(s0)

                @pl.when(i + 4 < niter)
                def _():
                    issue_aux(i + 4, s0)

            phase(i0, 0, 3)
            phase(i0 + 1, 1, 0)
            phase(i0 + 2, 2, 1)
            phase(i0 + 3, 3, 2)
            return 0

        lax.fori_loop(0, (niter + 3) // 4, quad_body, 0)

        for c in range(4):
            @pl.when(niter >= c + 1)
            def _(c=c):
                wait_scatters(c)

        plsc.subcore_barrier()

        for h in range(ROWS_PER_TILE // CHUNK):
            r0 = sid * ROWS_PER_TILE + h * CHUNK
            pltpu.sync_copy(acc.at[pl.ds(r0, CHUNK)], rows_v[h % 2])
            pltpu.sync_copy(rows_v[h % 2],
                            out_hbm.at[pl.ds(cid * HALF_ROWS + r0, CHUNK)])

    return body(table, values, row2, col2, split)


def kernel(table, b_values, b_row_idx, b_col_idx):
    row2 = b_row_idx.astype(jnp.int32).reshape(NNZ_TOTAL // ISEG, ISEG)
    col2 = b_col_idx.astype(jnp.int32).reshape(NNZ_TOTAL // ISEG, ISEG)
    split = _tc_split_count(row2)
    return _sc_aggregate(table, b_values, row2, col2, split)
